# Initial kernel scaffold; baseline (speedup 1.0000x reference)
import functools

import jax
import jax.numpy as jnp
from jax import lax
from jax.experimental import pallas as pl
from jax.experimental.pallas import tpu as pltpu

_N = 3
_SEG = 128
_IN = 64
_OUT = 64
_W3 = _N * _SEG


def _body(x_ref, w_ref, o_ref):
    x = x_ref[...]
    t = x.shape[0]

    m = lax.broadcasted_iota(jnp.int32, (t, _W3), 1)
    seg_of_m = m // _N
    k_of_m = m - seg_of_m * _N

    def step(j, acc):
        xj = lax.dynamic_slice_in_dim(x, j, 1, axis=1)
        idf = jnp.floor((xj + 1.0) * (_SEG / 2.0))
        idf = jnp.clip(idf, 0.0, float(_SEG - 1))
        idi = idf.astype(jnp.int32)
        x_min = idf * (2.0 / _SEG) - 1.0
        u = (xj - x_min) * (_SEG / 2.0) * 2.0 - 1.0
        b0 = 0.5 * u * (u - 1.0)
        b1 = 1.0 - u * u
        b2 = 0.5 * u * (u + 1.0)
        bk = jnp.where(k_of_m == 0, b0, jnp.where(k_of_m == 1, b1, b2))
        a = jnp.where(seg_of_m == idi, bk, 0.0)
        wj = w_ref[:, j, :]
        return acc + lax.dot_general(
            a, wj, (((1,), (1,)), ((), ())),
            preferred_element_type=jnp.float32)

    acc = jnp.zeros((t, _OUT), jnp.float32)
    o_ref[...] = lax.fori_loop(0, _IN, step, acc)


@jax.jit
def kernel(x, w):
    batch = x.shape[0]
    t = 256
    grid = (batch // t,)
    return pl.pallas_call(
        _body,
        grid=grid,
        in_specs=[
            pl.BlockSpec((t, _IN), lambda i: (i, 0)),
            pl.BlockSpec((_OUT, _IN, _W3), lambda i: (0, 0, 0)),
        ],
        out_specs=pl.BlockSpec((t, _OUT), lambda i: (i, 0)),
        out_shape=jax.ShapeDtypeStruct((batch, _OUT), jnp.float32),
    )(x, w)


# speedup vs baseline: 63.7433x; 63.7433x over previous
import jax
import jax.numpy as jnp
from jax import lax
from jax.experimental import pallas as pl
from jax.experimental.pallas import tpu as pltpu

_N = 3
_SEG = 128
_IN = 64
_OUT = 64
_JG = 8


def _body(x_ref, w_ref, o_ref):
    xb = x_ref[...]
    t = xb.shape[0]
    seg_iota = lax.broadcasted_iota(jnp.int32, (t, _SEG), 1)

    acc = jnp.zeros((t, _OUT), jnp.float32)
    for j in range(_IN):
        xj = xb[:, j:j + 1]
        idf = jnp.floor((xj + 1.0) * (_SEG / 2.0))
        idf = jnp.clip(idf, 0.0, float(_SEG - 1))
        idi = idf.astype(jnp.int32)
        x_min = idf * (2.0 / _SEG) - 1.0
        u = (xj - x_min) * _SEG - 1.0
        b = (0.5 * u * (u - 1.0), 1.0 - u * u, 0.5 * u * (u + 1.0))
        hot = seg_iota == idi
        for p in range(_N):
            hp = jnp.where(hot, b[p], 0.0)
            wk = w_ref[p, j]
            acc = acc + lax.dot_general(
                hp, wk, (((1,), (0,)), ((), ())),
                preferred_element_type=jnp.float32)

    o_ref[...] = acc


@jax.jit
def kernel(x, w):
    batch = x.shape[0]
    t = 256
    wr = jnp.transpose(w.reshape(_OUT, _IN, _SEG, _N), (3, 1, 2, 0))
    grid = (batch // t,)
    return pl.pallas_call(
        _body,
        grid=grid,
        in_specs=[
            pl.BlockSpec((t, _IN), lambda i: (i, 0)),
            pl.BlockSpec((_N, _IN, _SEG, _OUT), lambda i: (0, 0, 0, 0)),
        ],
        out_specs=pl.BlockSpec((t, _OUT), lambda i: (i, 0)),
        out_shape=jax.ShapeDtypeStruct((batch, _OUT), jnp.float32),
    )(x, wr)
